# baseline (speedup 1.0000x reference)
import functools

import jax
import jax.numpy as jnp
from jax import lax
from jax.experimental import pallas as pl
from jax.experimental.pallas import tpu as pltpu
from jax.experimental.pallas import tpu_sc as plsc

B = 320000
D = 128
C = 128
L = 16
NW = 32
BW = B // NW
NFULL = BW // C
TAIL = BW - NFULL * C
NBUF = 3


@jax.jit
def _impl(x_user, x_movie, u_idx, m_idx):
    mesh = plsc.VectorSubcoreMesh(core_axis_name="c", subcore_axis_name="s")

    @functools.partial(
        pl.kernel,
        mesh=mesh,
        out_type=jax.ShapeDtypeStruct((B,), jnp.float32),
        scratch_types=[
            pltpu.VMEM((BW,), jnp.int32),
            pltpu.VMEM((BW,), jnp.int32),
            pltpu.VMEM((C, D), jnp.float32),
            pltpu.VMEM((C, D), jnp.float32),
            pltpu.VMEM((C, D), jnp.float32),
            pltpu.VMEM((C, D), jnp.float32),
            pltpu.VMEM((C, D), jnp.float32),
            pltpu.VMEM((C, D), jnp.float32),
            pltpu.VMEM((BW,), jnp.float32),
            pltpu.SemaphoreType.DMA,
            pltpu.SemaphoreType.DMA,
        ],
        compiler_params=pltpu.CompilerParams(needs_layout_passes=False),
    )
    def k(u_hbm, m_hbm, uidx_hbm, midx_hbm, out_hbm,
          uidx_v, midx_v, u0, m0, u1, m1, u2, m2, out_v, sem_u, sem_m):
        ubuf = (u0, u1, u2)
        mbuf = (m0, m1, m2)
        wid = lax.axis_index("c") * 16 + lax.axis_index("s")
        wbase = wid * BW
        pltpu.sync_copy(uidx_hbm.at[pl.ds(wbase, BW)], uidx_v)
        pltpu.sync_copy(midx_hbm.at[pl.ds(wbase, BW)], midx_v)

        lane = lax.iota(jnp.int32, L)

        def fire(i, b):
            pltpu.async_copy(
                u_hbm.at[uidx_v.at[pl.ds(i * C, C)]], ubuf[b], sem_u)
            pltpu.async_copy(
                m_hbm.at[midx_v.at[pl.ds(i * C, C)]], mbuf[b], sem_m)

        def drain(i, b):
            pltpu.make_async_copy(
                u_hbm.at[uidx_v.at[pl.ds(i * C, C)]], ubuf[b], sem_u).wait()
            pltpu.make_async_copy(
                m_hbm.at[midx_v.at[pl.ds(i * C, C)]], mbuf[b], sem_m).wait()

        def compute(i, ub, mb, n_rows):
            for g in range(n_rows // L):
                rows = lane + g * L

                def col_body(jj, acc):
                    for t in range(8):
                        cols = jnp.full((L,), jj * 8 + t, jnp.int32)
                        acc = acc + (plsc.load_gather(ub, [rows, cols]) *
                                     plsc.load_gather(mb, [rows, cols]))
                    return acc

                acc = lax.fori_loop(0, D // 8, col_body,
                                    jnp.zeros((L,), jnp.float32))
                out_v[pl.ds(i * C + g * L, L)] = acc

        for b in range(NBUF):
            fire(b, b)

        def ring_body(kk, _):
            base = NBUF * kk
            for b in range(NBUF):
                i = base + b
                drain(i, b)
                fire(i + NBUF, b)
                compute(i, ubuf[b], mbuf[b], C)
            return 0

        n_main = (NFULL - NBUF) // NBUF
        lax.fori_loop(0, n_main, ring_body, 0)

        i0 = n_main * NBUF
        drain(i0, 0)
        compute(i0, u0, m0, C)
        ut = u0.at[pl.ds(0, TAIL)]
        mt = m0.at[pl.ds(0, TAIL)]
        tidx_u = uidx_v.at[pl.ds(NFULL * C, TAIL)]
        tidx_m = midx_v.at[pl.ds(NFULL * C, TAIL)]
        pltpu.async_copy(u_hbm.at[tidx_u], ut, sem_u)
        pltpu.async_copy(m_hbm.at[tidx_m], mt, sem_m)
        drain(i0 + 1, 1)
        compute(i0 + 1, u1, m1, C)
        drain(i0 + 2, 2)
        compute(i0 + 2, u2, m2, C)
        pltpu.make_async_copy(u_hbm.at[tidx_u], ut, sem_u).wait()
        pltpu.make_async_copy(m_hbm.at[tidx_m], mt, sem_m).wait()
        compute(NFULL, u0, m0, TAIL)

        pltpu.sync_copy(out_v, out_hbm.at[pl.ds(wbase, BW)])

    return k(x_user, x_movie, u_idx, m_idx)


def kernel(x_user, x_movie, edge_label_index):
    idx = edge_label_index.astype(jnp.int32)
    return _impl(x_user, x_movie, idx[0], idx[1])
